# Initial kernel scaffold; baseline (speedup 1.0000x reference)
import jax
import jax.numpy as jnp
from jax.experimental import pallas as pl


def kernel(x, edge_index, batch, l0_w1, l0_b1, l0_w2, l0_b2, l0_gamma, l0_beta, l1_w1, l1_b1, l1_w2, l1_b2, l1_gamma, l1_beta, l2_w1, l2_b1, l2_w2, l2_b2, l2_gamma, l2_beta):
    raise NotImplementedError("write your pallas kernel here")


# speedup vs baseline: 6.6841x; 6.6841x over previous
import functools

import jax
import jax.numpy as jnp
from jax import lax
from jax.experimental import pallas as pl
from jax.experimental.pallas import tpu as pltpu
from jax.experimental.pallas import tpu_sc as plsc

N = 10000
E = 320000
G = 64

NC = 2
NS = 16
CH = 128
NCHUNK = E // CH
ROWS_PER_TILE = 624
ROWS_REM = N - NS * ROWS_PER_TILE


def _sc_segsum_body(D, stage, h_hbm, src_hbm, dst_hbm, zero_hbm, out_hbm,
                    acc, h_sh, src_idx, dst_idx, rows, gsem):
    cid = lax.axis_index("c")
    sid = lax.axis_index("s")
    wid = sid * NC + cid

    pltpu.sync_copy(zero_hbm.at[pl.ds(sid * ROWS_PER_TILE, ROWS_PER_TILE)],
                    acc.at[pl.ds(sid * ROWS_PER_TILE, ROWS_PER_TILE)])
    if stage:
        pltpu.sync_copy(h_hbm.at[pl.ds(sid * ROWS_PER_TILE, ROWS_PER_TILE)],
                        h_sh.at[pl.ds(sid * ROWS_PER_TILE, ROWS_PER_TILE)])

    @pl.when(sid == NS - 1)
    def _zero_tail():
        tl = pl.ds(NS * ROWS_PER_TILE, ROWS_REM)
        pltpu.sync_copy(zero_hbm.at[tl], acc.at[tl])
        if stage:
            pltpu.sync_copy(h_hbm.at[tl], h_sh.at[tl])

    plsc.subcore_barrier()

    gather_src = h_sh if stage else h_hbm
    nfull = NCHUNK // (NC * NS)

    def step(k, _):
        c = wid + k * (NC * NS)

        @pl.when(c < NCHUNK)
        def _():
            base = c * CH
            pltpu.sync_copy(src_hbm.at[pl.ds(base, CH)], src_idx.at[0])
            pltpu.sync_copy(dst_hbm.at[pl.ds(base, CH)], dst_idx.at[0])
            pltpu.async_copy(gather_src.at[src_idx.at[0]], rows.at[0],
                             gsem).wait()
            pltpu.sync_copy(rows.at[0], acc.at[dst_idx.at[0]], add=True)

        return None

    lax.fori_loop(0, nfull + 1, step, None)

    plsc.subcore_barrier()
    sl = pl.ds(sid * ROWS_PER_TILE, ROWS_PER_TILE)
    pltpu.sync_copy(acc.at[sl], out_hbm.at[cid].at[sl])

    @pl.when(sid == NS - 1)
    def _write_tail():
        tl = pl.ds(NS * ROWS_PER_TILE, ROWS_REM)
        pltpu.sync_copy(acc.at[tl], out_hbm.at[cid].at[tl])


def _sc_segment_sum(h, src, dst, zero, D):
    stage = D * N * 4 * 2 <= 7 * 1024 * 1024
    mesh = plsc.VectorSubcoreMesh(core_axis_name="c", subcore_axis_name="s",
                                  num_cores=NC, num_subcores=NS)
    return pl.kernel(
        functools.partial(_sc_segsum_body, D, stage),
        out_type=jax.ShapeDtypeStruct((NC, N, D), jnp.float32),
        mesh=mesh,
        scratch_types=[
            pltpu.VMEM_SHARED((N, D), jnp.float32),
            pltpu.VMEM_SHARED((N, D) if stage else (8, D), jnp.float32),
            pltpu.VMEM((1, CH), jnp.int32),
            pltpu.VMEM((1, CH), jnp.int32),
            pltpu.VMEM((1, CH, D), jnp.float32),
            pltpu.SemaphoreType.DMA,
        ],
    )(h, src, dst, zero)


def _tc_dense_body(h_ref, p_ref, w1_ref, b1_ref, w2_ref, b2_ref,
                   g_ref, bt_ref, o_ref):
    z = h_ref[...] + p_ref[0] + p_ref[1]
    a = jnp.dot(z, w1_ref[...], preferred_element_type=jnp.float32)
    a = jnp.maximum(a + b1_ref[...], 0.0)
    hh = jnp.dot(a, w2_ref[...], preferred_element_type=jnp.float32)
    hh = hh + b2_ref[...]
    mean = jnp.mean(hh, axis=0, keepdims=True)
    var = jnp.mean((hh - mean) ** 2, axis=0, keepdims=True)
    hn = g_ref[...] * (hh - mean) * lax.rsqrt(var + 1e-5) + bt_ref[...]
    o_ref[...] = jnp.maximum(hn, 0.0)


def _tc_dense(h, p, w1, b1, w2, b2, gamma, beta, dout):
    return pl.pallas_call(
        _tc_dense_body,
        out_shape=jax.ShapeDtypeStruct((N, dout), jnp.float32),
    )(h, p, w1, b1, w2, b2, gamma, beta)


def _tc_dense_pool_body(h_ref, p_ref, w1_ref, b1_ref, w2_ref, b2_ref,
                        g_ref, bt_ref, batch_ref, o_ref):
    z = h_ref[...] + p_ref[0] + p_ref[1]
    a = jnp.dot(z, w1_ref[...], preferred_element_type=jnp.float32)
    a = jnp.maximum(a + b1_ref[...], 0.0)
    hh = jnp.dot(a, w2_ref[...], preferred_element_type=jnp.float32)
    hh = hh + b2_ref[...]
    mean = jnp.mean(hh, axis=0, keepdims=True)
    var = jnp.mean((hh - mean) ** 2, axis=0, keepdims=True)
    hn = g_ref[...] * (hh - mean) * lax.rsqrt(var + 1e-5) + bt_ref[...]
    hr = jnp.maximum(hn, 0.0)
    gids = lax.broadcasted_iota(jnp.int32, (N, G), 1)
    onehot = (batch_ref[...] == gids).astype(jnp.float32)
    sums = lax.dot_general(onehot, hr, (((0,), (0,)), ((), ())),
                           preferred_element_type=jnp.float32)
    ones = jnp.full((N, 1), 1.0, jnp.float32)
    counts = lax.dot_general(onehot, ones, (((0,), (0,)), ((), ())),
                             preferred_element_type=jnp.float32)
    o_ref[...] = sums / jnp.maximum(counts, 1.0)


def _tc_dense_pool(h, p, w1, b1, w2, b2, gamma, beta, batch2d, dout):
    return pl.pallas_call(
        _tc_dense_pool_body,
        out_shape=jax.ShapeDtypeStruct((G, dout), jnp.float32),
    )(h, p, w1, b1, w2, b2, gamma, beta, batch2d)


def kernel(x, edge_index, batch,
           l0_w1, l0_b1, l0_w2, l0_b2, l0_gamma, l0_beta,
           l1_w1, l1_b1, l1_w2, l1_b2, l1_gamma, l1_beta,
           l2_w1, l2_b1, l2_w2, l2_b2, l2_gamma, l2_beta):
    src = edge_index[0]
    dst = edge_index[1]
    batch2d = batch.reshape(N, 1)

    zero128 = jnp.zeros((N, 128), jnp.float32)
    zero64 = jnp.zeros((N, 64), jnp.float32)

    p0 = _sc_segment_sum(x, src, dst, zero128, 128)
    h1 = _tc_dense(x, p0, l0_w1, l0_b1.reshape(1, -1), l0_w2,
                   l0_b2.reshape(1, -1), l0_gamma.reshape(1, -1),
                   l0_beta.reshape(1, -1), 64)
    p1 = _sc_segment_sum(h1, src, dst, zero64, 64)
    h2 = _tc_dense(h1, p1, l1_w1, l1_b1.reshape(1, -1), l1_w2,
                   l1_b2.reshape(1, -1), l1_gamma.reshape(1, -1),
                   l1_beta.reshape(1, -1), 64)
    p2 = _sc_segment_sum(h2, src, dst, zero64, 64)
    out = _tc_dense_pool(h2, p2, l2_w1, l2_b1.reshape(1, -1), l2_w2,
                         l2_b2.reshape(1, -1), l2_gamma.reshape(1, -1),
                         l2_beta.reshape(1, -1), batch2d, 32)
    return out


# speedup vs baseline: 8.3465x; 1.2487x over previous
import functools

import jax
import jax.numpy as jnp
from jax import lax
from jax.experimental import pallas as pl
from jax.experimental.pallas import tpu as pltpu
from jax.experimental.pallas import tpu_sc as plsc

N = 10000
E = 320000
G = 64

NC = 2
NS = 16
CH = 128
NCHUNK = E // CH
ROWS_PER_TILE = 624
ROWS_REM = N - NS * ROWS_PER_TILE


def _sc_segsum_body(D, stage, nbuf, h_hbm, src_hbm, dst_hbm, zero_hbm,
                    out_hbm, acc, h_sh, idx_s0, idx_s1, idx_d0, idx_d1,
                    rows0, rows1, gsem0, gsem1, ssem0, ssem1):
    idx_s = [idx_s0, idx_s1]
    idx_d = [idx_d0, idx_d1]
    rows = [rows0, rows1]
    gsem = [gsem0, gsem1]
    ssem = [ssem0, ssem1]
    cid = lax.axis_index("c")
    sid = lax.axis_index("s")
    wid = sid * NC + cid

    pltpu.sync_copy(zero_hbm.at[pl.ds(sid * ROWS_PER_TILE, ROWS_PER_TILE)],
                    acc.at[pl.ds(sid * ROWS_PER_TILE, ROWS_PER_TILE)])
    if stage:
        pltpu.sync_copy(h_hbm.at[pl.ds(sid * ROWS_PER_TILE, ROWS_PER_TILE)],
                        h_sh.at[pl.ds(sid * ROWS_PER_TILE, ROWS_PER_TILE)])

    @pl.when(sid == NS - 1)
    def _zero_tail():
        tl = pl.ds(NS * ROWS_PER_TILE, ROWS_REM)
        pltpu.sync_copy(zero_hbm.at[tl], acc.at[tl])
        if stage:
            pltpu.sync_copy(h_hbm.at[tl], h_sh.at[tl])

    plsc.subcore_barrier()

    gather_src = h_sh if stage else h_hbm

    nwaves = NCHUNK // (NC * NS * nbuf) + 1

    def wave(k, _):
        gd = []
        for b in range(nbuf):
            c = wid + (k * nbuf + b) * (NC * NS)

            @pl.when(c < NCHUNK)
            def _fire(c=c, b=b):
                base = c * CH
                pltpu.sync_copy(src_hbm.at[pl.ds(base, CH)], idx_s[b].at[0])
                pltpu.sync_copy(dst_hbm.at[pl.ds(base, CH)], idx_d[b].at[0])
                gd.append(pltpu.async_copy(gather_src.at[idx_s[b].at[0]],
                                           rows[b].at[0], gsem[b]))

        for b in range(nbuf):
            c = wid + (k * nbuf + b) * (NC * NS)

            @pl.when(c < NCHUNK)
            def _drain(c=c, b=b):
                gd.pop(0).wait()
                pltpu.sync_copy(rows[b].at[0], acc.at[idx_d[b].at[0]],
                                add=True)

        return None

    lax.fori_loop(0, nwaves, wave, None)

    plsc.subcore_barrier()
    sl = pl.ds(sid * ROWS_PER_TILE, ROWS_PER_TILE)
    pltpu.sync_copy(acc.at[sl], out_hbm.at[cid].at[sl])

    @pl.when(sid == NS - 1)
    def _write_tail():
        tl = pl.ds(NS * ROWS_PER_TILE, ROWS_REM)
        pltpu.sync_copy(acc.at[tl], out_hbm.at[cid].at[tl])


def _sc_segment_sum(h, src, dst, zero, D):
    stage = D * N * 4 * 2 <= 7 * 1024 * 1024
    nbuf = 2
    mesh = plsc.VectorSubcoreMesh(core_axis_name="c", subcore_axis_name="s",
                                  num_cores=NC, num_subcores=NS)
    return pl.kernel(
        functools.partial(_sc_segsum_body, D, stage, nbuf),
        out_type=jax.ShapeDtypeStruct((NC, N, D), jnp.float32),
        mesh=mesh,
        scratch_types=[
            pltpu.VMEM_SHARED((N, D), jnp.float32),
            pltpu.VMEM_SHARED((N, D) if stage else (8, D), jnp.float32),
            pltpu.VMEM((1, CH), jnp.int32),
            pltpu.VMEM((1, CH), jnp.int32),
            pltpu.VMEM((1, CH), jnp.int32),
            pltpu.VMEM((1, CH), jnp.int32),
            pltpu.VMEM((1, CH, D), jnp.float32),
            pltpu.VMEM((1, CH, D), jnp.float32),
            pltpu.SemaphoreType.DMA,
            pltpu.SemaphoreType.DMA,
            pltpu.SemaphoreType.DMA,
            pltpu.SemaphoreType.DMA,
        ],
    )(h, src, dst, zero)


def _tc_dense_body(h_ref, p_ref, w1_ref, b1_ref, w2_ref, b2_ref,
                   g_ref, bt_ref, o_ref):
    z = h_ref[...] + p_ref[0] + p_ref[1]
    a = jnp.dot(z, w1_ref[...], preferred_element_type=jnp.float32)
    a = jnp.maximum(a + b1_ref[...], 0.0)
    hh = jnp.dot(a, w2_ref[...], preferred_element_type=jnp.float32)
    hh = hh + b2_ref[...]
    mean = jnp.mean(hh, axis=0, keepdims=True)
    var = jnp.mean((hh - mean) ** 2, axis=0, keepdims=True)
    hn = g_ref[...] * (hh - mean) * lax.rsqrt(var + 1e-5) + bt_ref[...]
    o_ref[...] = jnp.maximum(hn, 0.0)


def _tc_dense(h, p, w1, b1, w2, b2, gamma, beta, dout):
    return pl.pallas_call(
        _tc_dense_body,
        out_shape=jax.ShapeDtypeStruct((N, dout), jnp.float32),
    )(h, p, w1, b1, w2, b2, gamma, beta)


def _tc_dense_pool_body(h_ref, p_ref, w1_ref, b1_ref, w2_ref, b2_ref,
                        g_ref, bt_ref, batch_ref, o_ref):
    z = h_ref[...] + p_ref[0] + p_ref[1]
    a = jnp.dot(z, w1_ref[...], preferred_element_type=jnp.float32)
    a = jnp.maximum(a + b1_ref[...], 0.0)
    hh = jnp.dot(a, w2_ref[...], preferred_element_type=jnp.float32)
    hh = hh + b2_ref[...]
    mean = jnp.mean(hh, axis=0, keepdims=True)
    var = jnp.mean((hh - mean) ** 2, axis=0, keepdims=True)
    hn = g_ref[...] * (hh - mean) * lax.rsqrt(var + 1e-5) + bt_ref[...]
    hr = jnp.maximum(hn, 0.0)
    gids = lax.broadcasted_iota(jnp.int32, (N, G), 1)
    onehot = (batch_ref[...] == gids).astype(jnp.float32)
    sums = lax.dot_general(onehot, hr, (((0,), (0,)), ((), ())),
                           preferred_element_type=jnp.float32)
    ones = jnp.full((N, 1), 1.0, jnp.float32)
    counts = lax.dot_general(onehot, ones, (((0,), (0,)), ((), ())),
                             preferred_element_type=jnp.float32)
    o_ref[...] = sums / jnp.maximum(counts, 1.0)


def _tc_dense_pool(h, p, w1, b1, w2, b2, gamma, beta, batch2d, dout):
    return pl.pallas_call(
        _tc_dense_pool_body,
        out_shape=jax.ShapeDtypeStruct((G, dout), jnp.float32),
    )(h, p, w1, b1, w2, b2, gamma, beta, batch2d)


def kernel(x, edge_index, batch,
           l0_w1, l0_b1, l0_w2, l0_b2, l0_gamma, l0_beta,
           l1_w1, l1_b1, l1_w2, l1_b2, l1_gamma, l1_beta,
           l2_w1, l2_b1, l2_w2, l2_b2, l2_gamma, l2_beta):
    src = edge_index[0]
    dst = edge_index[1]
    batch2d = batch.reshape(N, 1)

    zero128 = jnp.zeros((N, 128), jnp.float32)
    zero64 = jnp.zeros((N, 64), jnp.float32)

    p0 = _sc_segment_sum(x, src, dst, zero128, 128)
    h1 = _tc_dense(x, p0, l0_w1, l0_b1.reshape(1, -1), l0_w2,
                   l0_b2.reshape(1, -1), l0_gamma.reshape(1, -1),
                   l0_beta.reshape(1, -1), 64)
    p1 = _sc_segment_sum(h1, src, dst, zero64, 64)
    h2 = _tc_dense(h1, p1, l1_w1, l1_b1.reshape(1, -1), l1_w2,
                   l1_b2.reshape(1, -1), l1_gamma.reshape(1, -1),
                   l1_beta.reshape(1, -1), 64)
    p2 = _sc_segment_sum(h2, src, dst, zero64, 64)
    out = _tc_dense_pool(h2, p2, l2_w1, l2_b1.reshape(1, -1), l2_w2,
                         l2_b2.reshape(1, -1), l2_gamma.reshape(1, -1),
                         l2_beta.reshape(1, -1), batch2d, 32)
    return out
